# baseline (speedup 1.0000x reference)
import jax
import jax.numpy as jnp
from jax.experimental import pallas as pl
from jax.experimental.pallas import tpu as pltpu

N_NODES = 10000
N_EDGES = 320000
D_H = 128
BE = 6400
BR = 8
BC = 800
NR = N_EDGES // BC
NEG = -1e30


def _dense_body(x_ref, w_ref, as_ref, ad_ref, h_ref, als_ref, ald_ref, se_ref):
    h = jnp.dot(x_ref[...], w_ref[...], preferred_element_type=jnp.float32)
    h_ref[...] = h
    als = jnp.sum(h * as_ref[...], axis=1)
    ald = jnp.sum(h * ad_ref[...], axis=1)
    als_ref[...] = als
    ald_ref[...] = ald
    s = als + ald
    se_ref[...] = jnp.where(s >= 0, s, s * 0.2)


def _p1_body(ei_ref, m_ref, als_ref, ald_ref, se_ref, e_ref, emax_ref):
    pi = pl.program_id(0)

    @pl.when(pi == 0)
    def _():
        def init(n, c):
            emax_ref[n] = se_ref[n]
            return c
        jax.lax.fori_loop(0, N_NODES, init, 0)

    def row(r, c0):
        def body(c, _):
            i = r * BC + c
            s = ei_ref[0, i]
            d = ei_ref[1, i]
            ev = als_ref[s] + ald_ref[d]
            ev = jnp.where(ev >= 0, ev, ev * 0.2)
            ev = jnp.where(m_ref[r, c] > 0, ev, NEG)
            e_ref[r, c] = ev
            emax_ref[d] = jnp.maximum(emax_ref[d], ev)
            return _
        jax.lax.fori_loop(0, BC, body, 0)
        return c0
    jax.lax.fori_loop(0, BR, row, 0)


def _gather_d_body(ei_ref, v_ref, o_ref):
    def row(r, c0):
        def body(c, _):
            o_ref[r, c] = v_ref[ei_ref[1, r * BC + c]]
            return _
        jax.lax.fori_loop(0, BC, body, 0)
        return c0
    jax.lax.fori_loop(0, BR, row, 0)


def _p3_body(ei_ref, e_ref, emax_ref, p_ref, den_ref):
    pi = pl.program_id(0)

    @pl.when(pi == 0)
    def _():
        def init(n, c):
            den_ref[n] = 0.0
            return c
        jax.lax.fori_loop(0, N_NODES, init, 0)

    def row(r, c0):
        def body(c, _):
            d = ei_ref[1, r * BC + c]
            pv = jnp.exp(e_ref[r, c] - emax_ref[d])
            p_ref[r, c] = pv
            den_ref[d] = den_ref[d] + pv
            return _
        jax.lax.fori_loop(0, BC, body, 0)
        return c0
    jax.lax.fori_loop(0, BR, row, 0)


def _p5_body(ei_ref, p_ref2, iv_ref, h_ref, o_ref):
    pi = pl.program_id(0)

    @pl.when(pi == 0)
    def _():
        o_ref[...] = jnp.zeros_like(o_ref)

    def row(r, c0):
        def body(c, _):
            i = r * BC + c
            s = ei_ref[0, i]
            d = ei_ref[1, i]
            a = p_ref2[r, c] * iv_ref[d]
            o_ref[pl.ds(d, 1), :] = o_ref[pl.ds(d, 1), :] + a * h_ref[pl.ds(s, 1), :]
            return _
        jax.lax.fori_loop(0, BC, body, 0)
        return c0
    jax.lax.fori_loop(0, BR, row, 0)


def _mean_body(x_ref, o_ref):
    o_ref[...] = jnp.sum(x_ref[...], axis=0, keepdims=True) * (1.0 / N_NODES)


_smem_n1 = pl.BlockSpec((N_NODES,), lambda i: (0,), memory_space=pltpu.SMEM)
_smem_eb = pl.BlockSpec((BR, BC), lambda i: (i, 0), memory_space=pltpu.SMEM)
_smem_ei = pl.BlockSpec((2, BE), lambda i: (0, i), memory_space=pltpu.SMEM)
_vmem_h = pl.BlockSpec((N_NODES, D_H), lambda i: (0, 0))

_f32 = jnp.float32
_GRID = (N_EDGES // BE,)

_dense = pl.pallas_call(
    _dense_body,
    out_shape=[
        jax.ShapeDtypeStruct((N_NODES, D_H), _f32),
        jax.ShapeDtypeStruct((N_NODES,), _f32),
        jax.ShapeDtypeStruct((N_NODES,), _f32),
        jax.ShapeDtypeStruct((N_NODES,), _f32),
    ],
)

_p1 = pl.pallas_call(
    _p1_body,
    grid=_GRID,
    in_specs=[_smem_ei, _smem_eb, _smem_n1, _smem_n1, _smem_n1],
    out_specs=[_smem_eb, _smem_n1],
    out_shape=[
        jax.ShapeDtypeStruct((NR, BC), _f32),
        jax.ShapeDtypeStruct((N_NODES,), _f32),
    ],
)

_gather_d = pl.pallas_call(
    _gather_d_body,
    grid=_GRID,
    in_specs=[_smem_ei, _smem_n1],
    out_specs=_smem_eb,
    out_shape=jax.ShapeDtypeStruct((NR, BC), _f32),
)

_p3 = pl.pallas_call(
    _p3_body,
    grid=_GRID,
    in_specs=[_smem_ei, _smem_eb, _smem_n1],
    out_specs=[_smem_eb, _smem_n1],
    out_shape=[
        jax.ShapeDtypeStruct((NR, BC), _f32),
        jax.ShapeDtypeStruct((N_NODES,), _f32),
    ],
)

_p5 = pl.pallas_call(
    _p5_body,
    grid=_GRID,
    in_specs=[_smem_ei, _smem_eb, _smem_n1, _vmem_h],
    out_specs=_vmem_h,
    out_shape=jax.ShapeDtypeStruct((N_NODES, D_H), _f32),
)

_mean = pl.pallas_call(
    _mean_body,
    out_shape=jax.ShapeDtypeStruct((1, D_H), _f32),
)


def _gat_layer(h_in, ei, mask_f, W, a_s, a_d, b):
    h, als, ald, se = _dense(h_in, W, a_s.reshape(1, -1), a_d.reshape(1, -1))
    e, emax = _p1(ei, mask_f.reshape(NR, BC), als, ald, se)
    p, den_e = _p3(ei, e, emax)
    p_self = jnp.exp(se - emax)
    denom = jnp.maximum(den_e + p_self, 1e-16)
    inv_den = 1.0 / denom
    out_e = _p5(ei, p, inv_den, h)
    return out_e + (p_self * inv_den)[:, None] * h + b[None, :]


def kernel(x, edge_index, batch, coord, W0, a_src0, a_dst0, b0, W1, a_src1, a_dst1, b1, W2, a_src2, a_dst2, b2, W3, a_src3, a_dst3, b3):
    src, dst = edge_index[0], edge_index[1]
    n = x.shape[0]
    eid = src * n + dst
    order = jnp.argsort(eid)
    se_sorted = eid[order]
    first_sorted = jnp.concatenate(
        [jnp.array([True]), se_sorted[1:] != se_sorted[:-1]])
    first = jnp.zeros((src.shape[0],), bool).at[order].set(first_sorted)
    cmask = (first & (src != dst)).astype(_f32)
    mask0 = (src != dst).astype(_f32)

    params = [
        (W0, a_src0, a_dst0, b0),
        (W1, a_src1, a_dst1, b1),
        (W2, a_src2, a_dst2, b2),
        (W3, a_src3, a_dst3, b3),
    ]
    h = jax.nn.relu(_gat_layer(x, edge_index, mask0, *params[0]))
    h = jax.nn.relu(_gat_layer(h, edge_index, cmask, *params[1]))
    h = jax.nn.relu(_gat_layer(h, edge_index, cmask, *params[2]))
    node_emb = _gat_layer(h, edge_index, cmask, *params[3])
    return _mean(node_emb)


# speedup vs baseline: 1.5705x; 1.5705x over previous
import jax
import jax.numpy as jnp
from jax.experimental import pallas as pl
from jax.experimental.pallas import tpu as pltpu

N_NODES = 10000
N_EDGES = 320000
D_H = 128
BE = 6400
BR = 8
BC = 800
NR = N_EDGES // BC
NEG = -1e30


def _dense_body(x_ref, w_ref, as_ref, ad_ref, h_ref, als_ref, ald_ref, se_ref):
    h = jnp.dot(x_ref[...], w_ref[...], preferred_element_type=jnp.float32)
    h_ref[...] = h
    als = jnp.sum(h * as_ref[...], axis=1)
    ald = jnp.sum(h * ad_ref[...], axis=1)
    als_ref[...] = als
    ald_ref[...] = ald
    s = als + ald
    se_ref[...] = jnp.where(s >= 0, s, s * 0.2)


def _p1_body(ei_ref, m_ref, als_ref, ald_ref, se_ref, e_ref, emax_ref):
    pi = pl.program_id(0)

    @pl.when(pi == 0)
    def _():
        def init(n, c):
            emax_ref[n] = se_ref[n]
            return c
        jax.lax.fori_loop(0, N_NODES, init, 0)

    def row(r, c0):
        def body(c, _):
            i = r * BC + c
            s = ei_ref[0, i]
            d = ei_ref[1, i]
            ev = als_ref[s] + ald_ref[d]
            ev = jnp.where(ev >= 0, ev, ev * 0.2)
            ev = jnp.where(m_ref[r, c] > 0, ev, NEG)
            e_ref[r, c] = ev
            emax_ref[d] = jnp.maximum(emax_ref[d], ev)
            return _
        jax.lax.fori_loop(0, BC, body, 0)
        return c0
    jax.lax.fori_loop(0, BR, row, 0)


def _gather_d_body(ei_ref, v_ref, o_ref):
    def row(r, c0):
        def body(c, _):
            o_ref[r, c] = v_ref[ei_ref[1, r * BC + c]]
            return _
        jax.lax.fori_loop(0, BC, body, 0)
        return c0
    jax.lax.fori_loop(0, BR, row, 0)


def _p3_body(ei_ref, p_ref, den_ref):
    pi = pl.program_id(0)

    @pl.when(pi == 0)
    def _():
        def init(n, c):
            den_ref[n] = 0.0
            return c
        jax.lax.fori_loop(0, N_NODES, init, 0)

    def row(r, c0):
        def body(c, _):
            d = ei_ref[1, r * BC + c]
            den_ref[d] = den_ref[d] + p_ref[r, c]
            return _
        jax.lax.fori_loop(0, BC, body, 0)
        return c0
    jax.lax.fori_loop(0, BR, row, 0)


def _p5_body(ei_ref, p_ref2, iv_ref, h_ref, o_ref):
    pi = pl.program_id(0)

    @pl.when(pi == 0)
    def _():
        o_ref[...] = jnp.zeros_like(o_ref)

    def row(r, c0):
        def body(c, _):
            i = r * BC + c
            s = ei_ref[0, i]
            d = ei_ref[1, i]
            a = p_ref2[r, c] * iv_ref[d]
            o_ref[pl.ds(d, 1), :] = o_ref[pl.ds(d, 1), :] + a * h_ref[pl.ds(s, 1), :]
            return _
        jax.lax.fori_loop(0, BC, body, 0)
        return c0
    jax.lax.fori_loop(0, BR, row, 0)


def _mean_body(x_ref, o_ref):
    o_ref[...] = jnp.sum(x_ref[...], axis=0, keepdims=True) * (1.0 / N_NODES)


_smem_n1 = pl.BlockSpec((N_NODES,), lambda i: (0,), memory_space=pltpu.SMEM)
_smem_eb = pl.BlockSpec((BR, BC), lambda i: (i, 0), memory_space=pltpu.SMEM)
_smem_ei = pl.BlockSpec((2, BE), lambda i: (0, i), memory_space=pltpu.SMEM)
_vmem_h = pl.BlockSpec((N_NODES, D_H), lambda i: (0, 0))

_f32 = jnp.float32
_GRID = (N_EDGES // BE,)

_dense = pl.pallas_call(
    _dense_body,
    out_shape=[
        jax.ShapeDtypeStruct((N_NODES, D_H), _f32),
        jax.ShapeDtypeStruct((N_NODES,), _f32),
        jax.ShapeDtypeStruct((N_NODES,), _f32),
        jax.ShapeDtypeStruct((N_NODES,), _f32),
    ],
)

_p1 = pl.pallas_call(
    _p1_body,
    grid=_GRID,
    in_specs=[_smem_ei, _smem_eb, _smem_n1, _smem_n1, _smem_n1],
    out_specs=[_smem_eb, _smem_n1],
    out_shape=[
        jax.ShapeDtypeStruct((NR, BC), _f32),
        jax.ShapeDtypeStruct((N_NODES,), _f32),
    ],
)

_gather_d = pl.pallas_call(
    _gather_d_body,
    grid=_GRID,
    in_specs=[_smem_ei, _smem_n1],
    out_specs=_smem_eb,
    out_shape=jax.ShapeDtypeStruct((NR, BC), _f32),
)

_p3 = pl.pallas_call(
    _p3_body,
    grid=_GRID,
    in_specs=[_smem_ei, _smem_eb],
    out_specs=_smem_n1,
    out_shape=jax.ShapeDtypeStruct((N_NODES,), _f32),
)

_p5 = pl.pallas_call(
    _p5_body,
    grid=_GRID,
    in_specs=[_smem_ei, _smem_eb, _smem_n1, _vmem_h],
    out_specs=_vmem_h,
    out_shape=jax.ShapeDtypeStruct((N_NODES, D_H), _f32),
)

_mean = pl.pallas_call(
    _mean_body,
    out_shape=jax.ShapeDtypeStruct((1, D_H), _f32),
)


def _gat_layer(h_in, ei, mask_f, W, a_s, a_d, b):
    h, als, ald, se = _dense(h_in, W, a_s.reshape(1, -1), a_d.reshape(1, -1))
    e, emax = _p1(ei, mask_f.reshape(NR, BC), als, ald, se)
    em_g = _gather_d(ei, emax)
    p = jnp.exp(e - em_g)
    den_e = _p3(ei, p)
    p_self = jnp.exp(se - emax)
    denom = jnp.maximum(den_e + p_self, 1e-16)
    inv_den = 1.0 / denom
    out_e = _p5(ei, p, inv_den, h)
    return out_e + (p_self * inv_den)[:, None] * h + b[None, :]


def kernel(x, edge_index, batch, coord, W0, a_src0, a_dst0, b0, W1, a_src1, a_dst1, b1, W2, a_src2, a_dst2, b2, W3, a_src3, a_dst3, b3):
    src, dst = edge_index[0], edge_index[1]
    n = x.shape[0]
    eid = src * n + dst
    order = jnp.argsort(eid)
    se_sorted = eid[order]
    first_sorted = jnp.concatenate(
        [jnp.array([True]), se_sorted[1:] != se_sorted[:-1]])
    first = jnp.zeros((src.shape[0],), bool).at[order].set(first_sorted)
    cmask = (first & (src != dst)).astype(_f32)
    mask0 = (src != dst).astype(_f32)

    params = [
        (W0, a_src0, a_dst0, b0),
        (W1, a_src1, a_dst1, b1),
        (W2, a_src2, a_dst2, b2),
        (W3, a_src3, a_dst3, b3),
    ]
    h = jax.nn.relu(_gat_layer(x, edge_index, mask0, *params[0]))
    h = jax.nn.relu(_gat_layer(h, edge_index, cmask, *params[1]))
    h = jax.nn.relu(_gat_layer(h, edge_index, cmask, *params[2]))
    node_emb = _gat_layer(h, edge_index, cmask, *params[3])
    return _mean(node_emb)
